# baseline (speedup 1.0000x reference)
import functools

import jax
import jax.numpy as jnp
from jax import lax
from jax.experimental import pallas as pl
from jax.experimental.pallas import tpu as pltpu
from jax.experimental.pallas import tpu_sc as plsc

N_PRE = 100000
N_POST = 100000
KK = 32
RR = 16
BB = 8
L = 16
NW = 32
ROWS_BLK = 1664
NBLK = 2 * NW
N_POST_PAD = NBLK * ROWS_BLK
WIN = 1600
VTW = (RR // 2) * WIN
XW = BB * WIN
GROUPS = ROWS_BLK // L
NT = 2 * KK

_LOS = [min(max((j * (N_PRE - 1)) // (KK - 1) - 782, 0) & ~7, N_PRE - WIN)
        for j in range(KK)]


def _sc_body(vtw_hbm, xw_hbm, ut_hbm, idxt_hbm, y_hbm,
             vt_flat, x_flat, u_buf, idx2, y_buf, sem_a, sem_b):
    wid = lax.axis_index("s") * 2 + lax.axis_index("c")
    bi0 = wid * 2

    def _win_copies(t, parity, sem):
        j = t & (KK - 1)
        bi = bi0 + (t >> 5)
        return (
            pltpu.make_async_copy(
                vtw_hbm.at[pl.ds(j * VTW, VTW)],
                vt_flat.at[pl.ds(parity * VTW, VTW)], sem),
            pltpu.make_async_copy(
                xw_hbm.at[pl.ds(j * XW, XW)],
                x_flat.at[pl.ds(parity * XW, XW)], sem),
            pltpu.make_async_copy(
                idxt_hbm.at[pl.ds(j * N_POST_PAD + bi * ROWS_BLK, ROWS_BLK)],
                idx2.at[pl.ds(parity * ROWS_BLK, ROWS_BLK)], sem),
        )

    def _win_start(t, parity, sem):
        for cp in _win_copies(t, parity, sem):
            cp.start()

    def _win_wait(t, parity, sem):
        for cp in _win_copies(t, parity, sem):
            cp.wait()

    def _zero_y():
        def zb(g, c):
            z = jnp.zeros((L,), jnp.float32)
            for b in range(BB):
                y_buf[b, pl.ds(g * L, L)] = z
            return c
        lax.fori_loop(0, GROUPS, zb, 0)

    def _load_u(bi):
        pltpu.sync_copy(ut_hbm.at[:, pl.ds(bi * ROWS_BLK, ROWS_BLK)], u_buf)

    def _flush_y(bi):
        pltpu.sync_copy(y_buf, y_hbm.at[:, pl.ds(bi * ROWS_BLK, ROWS_BLK)])

    def _compute(t, parity):
        j = t & (KK - 1)
        base_floor = (j * (N_PRE - 1)) // (KK - 1)
        lo = jnp.minimum(
            jnp.maximum(base_floor - 782, 0) & (-8), N_PRE - WIN)

        @plsc.parallel_loop(0, GROUPS, 1, unroll=1)
        def g_body(g):
            g16 = g * L
            iv = idx2[pl.ds(parity * ROWS_BLK + g16, L)]
            il = jnp.minimum(jnp.maximum(iv - lo, 0), WIN - 1)
            ilv = il + (parity * VTW)
            ilx = il + (parity * XW)
            prods = []
            for rr in range(RR // 2):
                w = plsc.load_gather(vt_flat, [ilv + (rr * WIN)])
                v0, v1 = plsc.unpack(
                    plsc.bitcast(w, jnp.bfloat16),
                    format=plsc.PackFormat.INTERLEAVED)
                prods.append(v0 * u_buf[2 * rr, pl.ds(g16, L)])
                prods.append(v1 * u_buf[2 * rr + 1, pl.ds(g16, L)])
            while len(prods) > 1:
                prods = [a + b for a, b in zip(prods[::2], prods[1::2])]
            val = prods[0]
            xs = [plsc.load_gather(x_flat, [ilx + (b * WIN)])
                  for b in range(BB)]
            ps = [xb * val for xb in xs]
            for b in range(BB):
                plsc.addupdate(y_buf.at[b, pl.ds(g16, L)], ps[b])

    _load_u(bi0)
    _zero_y()
    _win_start(0, 0, sem_a)

    def m_body(m, c):
        t0 = 2 * m
        t1 = t0 + 1

        @pl.when(t0 == KK)
        def _pass_switch():
            _flush_y(bi0)
            _load_u(bi0 + 1)
            _zero_y()

        _win_start(t1, 1, sem_b)
        _win_wait(t0, 0, sem_a)
        _compute(t0, 0)

        @pl.when(t1 < NT - 1)
        def _prefetch_even():
            _win_start(t1 + 1, 0, sem_a)

        _win_wait(t1, 1, sem_b)
        _compute(t1, 1)
        return c

    lax.fori_loop(0, NT // 2, m_body, 0)
    _flush_y(bi0 + 1)


_sc_call = functools.partial(
    pl.kernel,
    out_type=jax.ShapeDtypeStruct((BB, N_POST_PAD), jnp.float32),
    mesh=plsc.VectorSubcoreMesh(core_axis_name="c", subcore_axis_name="s"),
    compiler_params=pltpu.CompilerParams(
        use_tc_tiling_on_sc=False, needs_layout_passes=False),
    scratch_types=[
        pltpu.VMEM((2 * VTW,), jnp.int32),
        pltpu.VMEM((2 * XW,), jnp.float32),
        pltpu.VMEM((RR, ROWS_BLK), jnp.float32),
        pltpu.VMEM((2 * ROWS_BLK,), jnp.int32),
        pltpu.VMEM((BB, ROWS_BLK), jnp.float32),
        pltpu.SemaphoreType.DMA,
        pltpu.SemaphoreType.DMA,
    ],
)(_sc_body)


def _pack_rows(a):
    a16 = jax.lax.bitcast_convert_type(a.astype(jnp.bfloat16), jnp.uint16)
    return jax.lax.bitcast_convert_type(
        a16[0::2].astype(jnp.uint32) | (a16[1::2].astype(jnp.uint32) << 16),
        jnp.int32)


@jax.jit
def _run(x, U, V, indices):
    pad_n = N_POST_PAD - N_POST
    vwords = _pack_rows(V.T)
    vtwf = jnp.concatenate(
        [vwords[:, lo:lo + WIN].reshape(-1) for lo in _LOS])
    xwf = jnp.concatenate(
        [x[:, lo:lo + WIN].reshape(-1) for lo in _LOS])
    ut = jnp.pad(U.T, ((0, 0), (0, pad_n)))
    idxt = jnp.pad(indices.reshape(N_POST, KK).T,
                   ((0, 0), (0, pad_n))).reshape(-1)
    ypad = _sc_call(vtwf, xwf, ut, idxt)
    return ypad[:, :N_POST]


def kernel(x, U, V, indices):
    return _run(x, U, V, indices)


# speedup vs baseline: 1.0071x; 1.0071x over previous
import functools

import jax
import jax.numpy as jnp
from jax import lax
from jax.experimental import pallas as pl
from jax.experimental.pallas import tpu as pltpu
from jax.experimental.pallas import tpu_sc as plsc

N_PRE = 100000
N_POST = 100000
KK = 32
RR = 16
BB = 8
L = 16
NW = 32
ROWS_BLK = 1664
NBLK = 2 * NW
N_POST_PAD = NBLK * ROWS_BLK
WIN = 1600
VTW = (RR // 2) * WIN
XW = BB * WIN
GROUPS = ROWS_BLK // L
NT = 2 * KK

_LOS = [min(max((j * (N_PRE - 1)) // (KK - 1) - 782, 0) & ~7, N_PRE - WIN)
        for j in range(KK)]


def _sc_body(vtw_hbm, xw_hbm, ut_hbm, idxt_hbm, y_hbm,
             vt_flat, x_flat, u_buf, idx2, y_buf, sem_a, sem_b):
    wid = lax.axis_index("s") * 2 + lax.axis_index("c")
    bi0 = wid * 2

    def _win_copies(t, parity, sem):
        j = t & (KK - 1)
        bi = bi0 + (t >> 5)
        return (
            pltpu.make_async_copy(
                vtw_hbm.at[pl.ds(j * VTW, VTW)],
                vt_flat.at[pl.ds(parity * VTW, VTW)], sem),
            pltpu.make_async_copy(
                xw_hbm.at[pl.ds(j * XW, XW)],
                x_flat.at[pl.ds(parity * XW, XW)], sem),
            pltpu.make_async_copy(
                idxt_hbm.at[pl.ds(j * N_POST_PAD + bi * ROWS_BLK, ROWS_BLK)],
                idx2.at[pl.ds(parity * ROWS_BLK, ROWS_BLK)], sem),
        )

    def _win_start(t, parity, sem):
        for cp in _win_copies(t, parity, sem):
            cp.start()

    def _win_wait(t, parity, sem):
        for cp in _win_copies(t, parity, sem):
            cp.wait()

    def _zero_y():
        def zb(g, c):
            z = jnp.zeros((L,), jnp.float32)
            for b in range(BB):
                y_buf[b, pl.ds(g * L, L)] = z
            return c
        lax.fori_loop(0, GROUPS, zb, 0)

    def _load_u(bi):
        pltpu.sync_copy(ut_hbm.at[:, pl.ds(bi * ROWS_BLK, ROWS_BLK)], u_buf)

    def _flush_y(bi):
        pltpu.sync_copy(y_buf, y_hbm.at[:, pl.ds(bi * ROWS_BLK, ROWS_BLK)])

    def _compute(t, parity):
        j = t & (KK - 1)
        base_floor = (j * (N_PRE - 1)) // (KK - 1)
        lo = jnp.minimum(
            jnp.maximum(base_floor - 782, 0) & (-8), N_PRE - WIN)

        def g_body(g, cc):
            g16 = g * L
            iv = idx2[pl.ds(parity * ROWS_BLK + g16, L)]
            il = jnp.minimum(jnp.maximum(iv - lo, 0), WIN - 1)
            ilv = il + (parity * VTW)
            ilx = il + (parity * XW)
            prods = []
            for rr in range(RR // 2):
                w = plsc.load_gather(vt_flat, [ilv + (rr * WIN)])
                v0, v1 = plsc.unpack(
                    plsc.bitcast(w, jnp.bfloat16),
                    format=plsc.PackFormat.INTERLEAVED)
                prods.append(v0 * u_buf[2 * rr, pl.ds(g16, L)])
                prods.append(v1 * u_buf[2 * rr + 1, pl.ds(g16, L)])
            while len(prods) > 1:
                prods = [a + b for a, b in zip(prods[::2], prods[1::2])]
            val = prods[0]
            xs = [plsc.load_gather(x_flat, [ilx + (b * WIN)])
                  for b in range(BB)]
            ps = [xb * val for xb in xs]
            for b in range(BB):
                plsc.addupdate(y_buf.at[b, pl.ds(g16, L)], ps[b])
            return cc
        lax.fori_loop(0, GROUPS, g_body, 0)

    _load_u(bi0)
    _zero_y()
    _win_start(0, 0, sem_a)

    def m_body(m, c):
        t0 = 2 * m
        t1 = t0 + 1

        @pl.when(t0 == KK)
        def _pass_switch():
            _flush_y(bi0)
            _load_u(bi0 + 1)
            _zero_y()

        _win_start(t1, 1, sem_b)
        _win_wait(t0, 0, sem_a)
        _compute(t0, 0)

        @pl.when(t1 < NT - 1)
        def _prefetch_even():
            _win_start(t1 + 1, 0, sem_a)

        _win_wait(t1, 1, sem_b)
        _compute(t1, 1)
        return c

    lax.fori_loop(0, NT // 2, m_body, 0)
    _flush_y(bi0 + 1)


_sc_call = functools.partial(
    pl.kernel,
    out_type=jax.ShapeDtypeStruct((BB, N_POST_PAD), jnp.float32),
    mesh=plsc.VectorSubcoreMesh(core_axis_name="c", subcore_axis_name="s"),
    compiler_params=pltpu.CompilerParams(
        use_tc_tiling_on_sc=False, needs_layout_passes=False),
    scratch_types=[
        pltpu.VMEM((2 * VTW,), jnp.int32),
        pltpu.VMEM((2 * XW,), jnp.float32),
        pltpu.VMEM((RR, ROWS_BLK), jnp.float32),
        pltpu.VMEM((2 * ROWS_BLK,), jnp.int32),
        pltpu.VMEM((BB, ROWS_BLK), jnp.float32),
        pltpu.SemaphoreType.DMA,
        pltpu.SemaphoreType.DMA,
    ],
)(_sc_body)


def _pack_rows(a):
    a16 = jax.lax.bitcast_convert_type(a.astype(jnp.bfloat16), jnp.uint16)
    return jax.lax.bitcast_convert_type(
        a16[0::2].astype(jnp.uint32) | (a16[1::2].astype(jnp.uint32) << 16),
        jnp.int32)


@jax.jit
def _run(x, U, V, indices):
    pad_n = N_POST_PAD - N_POST
    vwords = _pack_rows(V.T)
    vtwf = jnp.concatenate(
        [vwords[:, lo:lo + WIN].reshape(-1) for lo in _LOS])
    xwf = jnp.concatenate(
        [x[:, lo:lo + WIN].reshape(-1) for lo in _LOS])
    ut = jnp.pad(U.T, ((0, 0), (0, pad_n)))
    idxt = jnp.pad(indices.reshape(N_POST, KK).T,
                   ((0, 0), (0, pad_n))).reshape(-1)
    ypad = _sc_call(vtwf, xwf, ut, idxt)
    return ypad[:, :N_POST]


def kernel(x, U, V, indices):
    return _run(x, U, V, indices)
